# baseline (speedup 1.0000x reference)
import functools

import jax
import jax.numpy as jnp
from jax import lax
from jax.experimental import pallas as pl
from jax.experimental.pallas import tpu as pltpu
from jax.experimental.pallas import tpu_sc as plsc

NC = 2
NS = 16
NW = NC * NS
L = 16

D = 128
K = 128
NBUF = 4


def _sc_lookup(idx2d, trainable, fixed):
    n_chunks_total, k = idx2d.shape
    assert k == K and n_chunks_total % NW == 0
    n_chunks = n_chunks_total // NW
    assert n_chunks % NBUF == 0
    b_total = n_chunks_total * K
    train_n = trainable.shape[0]
    mesh = plsc.VectorSubcoreMesh(core_axis_name="c", subcore_axis_name="s")

    slot_scratch = []
    for _ in range(NBUF):
        slot_scratch += [
            pltpu.VMEM((K,), jnp.int32),
            pltpu.VMEM((K, D), jnp.float32),
            pltpu.VMEM((K + L,), jnp.int32),
            pltpu.VMEM((K + L,), jnp.int32),
            pltpu.SemaphoreType.DMA,
            pltpu.SemaphoreType.DMA,
        ]

    @functools.partial(
        pl.kernel,
        out_type=jax.ShapeDtypeStruct((b_total, D), jnp.float32),
        mesh=mesh,
        scratch_types=[pltpu.VMEM((n_chunks, K), jnp.int32)] + slot_scratch
        + [pltpu.SemaphoreType.DMA],
        compiler_params=pltpu.CompilerParams(needs_layout_passes=False),
    )
    def k_fn(idx_hbm, train_hbm, fixed_hbm, out_hbm, idxall, *rest):
        slots = [tuple(rest[i * 6:(i + 1) * 6]) for i in range(NBUF)]
        psem = rest[NBUF * 6]
        wid = lax.axis_index("s") * NC + lax.axis_index("c")
        row0 = wid * (n_chunks * K)
        pltpu.sync_copy(idx_hbm.at[pl.ds(wid * n_chunks, n_chunks)], idxall)

        def front(c, s):
            fidx, buf, jl, tl, gsem, _ = slots[s]

            def grp(g, off):
                v = idxall[c, pl.ds(g * L, L)]
                is_tr = v < train_n
                fidx[pl.ds(g * L, L)] = jnp.maximum(v - train_n, 0)
                jvec = lax.iota(jnp.int32, L) + g * L
                pfx = plsc.cumsum(is_tr.astype(jnp.int32))
                lanes = off + pfx - 1
                plsc.store_scatter(jl, [lanes], jvec, mask=is_tr)
                plsc.store_scatter(tl, [lanes], v, mask=is_tr)
                return off + pfx[L - 1]

            n_tr = lax.fori_loop(0, K // L, grp, jnp.int32(0))
            pltpu.async_copy(fixed_hbm.at[fidx], buf, gsem)
            return n_tr

        def finish(s, base, n_tr):
            fidx, buf, jl, tl, gsem, wsem = slots[s]
            pltpu.make_async_copy(fixed_hbm.at[fidx], buf, gsem).wait()

            def patch_issue(i, _):
                j = jl[pl.ds(i, L)][0]
                t = tl[pl.ds(i, L)][0]
                pltpu.async_copy(train_hbm.at[t], buf.at[j], psem)
                return 0

            def patch_drain(i, _):
                pltpu.make_async_copy(train_hbm.at[0], buf.at[0], psem).wait()
                return 0

            lax.fori_loop(0, n_tr, patch_issue, 0)
            lax.fori_loop(0, n_tr, patch_drain, 0)
            pltpu.async_copy(buf, out_hbm.at[pl.ds(base, K)], wsem)

        def step(st, ntrs):
            ntrs = list(ntrs)
            for s in range(NBUF):
                c = st * NBUF + s
                buf_s, wsem_s = slots[s][1], slots[s][5]

                @pl.when(c >= NBUF)
                def _():
                    pltpu.make_async_copy(buf_s, out_hbm.at[pl.ds(row0, K)],
                                          wsem_s).wait()

                ntrs[s] = lax.cond(c < n_chunks, lambda c=c, s=s: front(c, s),
                                   lambda: jnp.int32(0))

                sf = (s + 1) % NBUF
                cf = c - (NBUF - 1)

                @pl.when((cf >= 0) & (cf < n_chunks))
                def _():
                    finish(sf, row0 + cf * K, ntrs[sf])

            return tuple(ntrs)

        lax.fori_loop(0, n_chunks // NBUF + 1, step,
                      (jnp.int32(0),) * NBUF)

    return k_fn(idx2d, trainable, fixed)


def kernel(indices, trainable_embedding, fixed_embedding):
    b, t = indices.shape
    idx2d = indices.reshape(-1, K).astype(jnp.int32)
    out = _sc_lookup(idx2d, trainable_embedding, fixed_embedding)
    return out.reshape(b, t, D)


# speedup vs baseline: 2.9272x; 2.9272x over previous
import functools

import jax
import jax.numpy as jnp
from jax import lax
from jax.experimental import pallas as pl
from jax.experimental.pallas import tpu as pltpu
from jax.experimental.pallas import tpu_sc as plsc

NC = 2
NS = 16
NW = NC * NS
L = 16

D = 128
K = 128
NBUF = 4


def _sc_lookup(idx2d, trainable, fixed):
    n_chunks_total, k = idx2d.shape
    assert k == K and n_chunks_total % NW == 0
    n_chunks = n_chunks_total // NW
    assert n_chunks % NBUF == 0
    b_total = n_chunks_total * K
    train_n = trainable.shape[0]
    mesh = plsc.VectorSubcoreMesh(core_axis_name="c", subcore_axis_name="s")

    slot_scratch = []
    for _ in range(NBUF):
        slot_scratch += [
            pltpu.VMEM((K,), jnp.int32),
            pltpu.VMEM((K, D), jnp.float32),
            pltpu.VMEM((K + L,), jnp.int32),
            pltpu.VMEM((K + L,), jnp.int32),
            pltpu.SemaphoreType.DMA,
            pltpu.SemaphoreType.DMA,
        ]

    @functools.partial(
        pl.kernel,
        out_type=jax.ShapeDtypeStruct((b_total, D), jnp.float32),
        mesh=mesh,
        scratch_types=[pltpu.VMEM((n_chunks, K), jnp.int32)] + slot_scratch
        + [pltpu.SemaphoreType.DMA],
        compiler_params=pltpu.CompilerParams(needs_layout_passes=False),
    )
    def k_fn(idx_hbm, train_hbm, fixed_hbm, out_hbm, idxall, *rest):
        slots = [tuple(rest[i * 6:(i + 1) * 6]) for i in range(NBUF)]
        psem = rest[NBUF * 6]
        wid = lax.axis_index("s") * NC + lax.axis_index("c")
        row0 = wid * (n_chunks * K)
        pltpu.sync_copy(idx_hbm.at[pl.ds(wid * n_chunks, n_chunks)], idxall)

        def front(c, s):
            fidx, buf, jl, tl, gsem, _ = slots[s]

            def grp(g, off):
                v = idxall[c, pl.ds(g * L, L)]
                is_tr = v < train_n
                jvec = lax.iota(jnp.int32, L) + g * L
                spread = (row0 + c * K + jvec) & 0xFFFF
                fidx[pl.ds(g * L, L)] = jnp.where(is_tr, spread, v - train_n)
                pfx = plsc.cumsum(is_tr.astype(jnp.int32))
                lanes = off + pfx - 1
                plsc.store_scatter(jl, [lanes], jvec, mask=is_tr)
                plsc.store_scatter(tl, [lanes], v, mask=is_tr)
                return off + pfx[L - 1]

            n_tr = lax.fori_loop(0, K // L, grp, jnp.int32(0))
            pltpu.async_copy(fixed_hbm.at[fidx], buf, gsem)
            return n_tr

        def finish(s, base, n_tr):
            fidx, buf, jl, tl, gsem, wsem = slots[s]
            pltpu.make_async_copy(fixed_hbm.at[fidx], buf, gsem).wait()

            def patch_issue(i, _):
                j = jl[pl.ds(i, L)][0]
                t = tl[pl.ds(i, L)][0]
                pltpu.async_copy(train_hbm.at[t], buf.at[j], psem)
                return 0

            def patch_drain(i, _):
                pltpu.make_async_copy(train_hbm.at[0], buf.at[0], psem).wait()
                return 0

            lax.fori_loop(0, n_tr, patch_issue, 0)
            lax.fori_loop(0, n_tr, patch_drain, 0)
            pltpu.async_copy(buf, out_hbm.at[pl.ds(base, K)], wsem)

        def step(st, ntrs):
            ntrs = list(ntrs)
            for s in range(NBUF):
                c = st * NBUF + s
                buf_s, wsem_s = slots[s][1], slots[s][5]

                @pl.when(c >= NBUF)
                def _():
                    pltpu.make_async_copy(buf_s, out_hbm.at[pl.ds(row0, K)],
                                          wsem_s).wait()

                ntrs[s] = lax.cond(c < n_chunks, lambda c=c, s=s: front(c, s),
                                   lambda: jnp.int32(0))

                sf = (s + 1) % NBUF
                cf = c - (NBUF - 1)

                @pl.when((cf >= 0) & (cf < n_chunks))
                def _():
                    finish(sf, row0 + cf * K, ntrs[sf])

            return tuple(ntrs)

        lax.fori_loop(0, n_chunks // NBUF + 1, step,
                      (jnp.int32(0),) * NBUF)

    return k_fn(idx2d, trainable, fixed)


def kernel(indices, trainable_embedding, fixed_embedding):
    b, t = indices.shape
    idx2d = indices.reshape(-1, K).astype(jnp.int32)
    out = _sc_lookup(idx2d, trainable_embedding, fixed_embedding)
    return out.reshape(b, t, D)
